# Initial kernel scaffold; baseline (speedup 1.0000x reference)
import functools

import jax
import jax.numpy as jnp
from jax.experimental import pallas as pl

N_HASHES = 4
CHUNK = 144
RES_SCALE = 1.0
REDUCTION = 4


def _conv(x, w, b):
    out = jax.lax.conv_general_dilated(x, w, (1, 1), 'SAME',
                                       dimension_numbers=('NCHW', 'OIHW', 'NCHW'))
    return out + b[None, :, None, None]


def _attn_kernel(xs_ref, ys_ref, out_ref, lse_ref, *, nc):
    k = pl.program_id(1)
    q = xs_ref[0, pl.ds(k * CHUNK, CHUNK), :]

    prev = jax.lax.rem(k - 1 + nc, nc)
    nxt = jax.lax.rem(k + 1, nc)

    def score_block(j):
        kx = xs_ref[0, pl.ds(j * CHUNK, CHUNK), :]
        nrm = jnp.sqrt(jnp.sum(kx * kx, axis=1, keepdims=True))
        kn = kx / jnp.maximum(nrm, 5e-05)
        return jax.lax.dot_general(q, kn, (((1,), (1,)), ((), ())),
                                   preferred_element_type=jnp.float32)

    s_c = score_block(k)
    s_p = score_block(prev)
    s_n = score_block(nxt)

    m = jnp.maximum(jnp.maximum(jnp.max(s_c, axis=1), jnp.max(s_p, axis=1)),
                    jnp.max(s_n, axis=1))[:, None]
    p_c = jnp.exp(s_c - m)
    p_p = jnp.exp(s_p - m)
    p_n = jnp.exp(s_n - m)
    denom = (jnp.sum(p_c, axis=1) + jnp.sum(p_p, axis=1)
             + jnp.sum(p_n, axis=1))[:, None]

    def pv(p, j):
        v = ys_ref[0, pl.ds(j * CHUNK, CHUNK), :]
        return jax.lax.dot_general(p, v, (((1,), (0,)), ((), ())),
                                   preferred_element_type=jnp.float32)

    acc = pv(p_c, k) + pv(p_p, prev) + pv(p_n, nxt)
    out_ref[0, :, :] = acc / denom
    lse_ref[0, :, :] = m + jnp.log(denom)


def _banded_attention(xs, ys):
    H, Lp, C = xs.shape
    Cr = ys.shape[-1]
    nc = Lp // CHUNK
    kern = functools.partial(_attn_kernel, nc=nc)
    out, lse = pl.pallas_call(
        kern,
        grid=(H, nc),
        in_specs=[
            pl.BlockSpec((1, Lp, C), lambda h, k: (h, 0, 0)),
            pl.BlockSpec((1, Lp, Cr), lambda h, k: (h, 0, 0)),
        ],
        out_specs=[
            pl.BlockSpec((1, CHUNK, Cr), lambda h, k: (h, k, 0)),
            pl.BlockSpec((1, CHUNK, 1), lambda h, k: (h, k, 0)),
        ],
        out_shape=[
            jax.ShapeDtypeStruct((H, Lp, Cr), jnp.float32),
            jax.ShapeDtypeStruct((H, Lp, 1), jnp.float32),
        ],
    )(xs, ys)
    return out, lse[..., 0]


def kernel(input, w_match, b_match, w_assembly, b_assembly):
    x = input
    N, _, Hh, Ww = x.shape
    L = Hh * Ww
    C = w_match.shape[0]
    Cr = w_assembly.shape[0]

    x_embed = _conv(x, w_match, b_match).reshape(N, C, L).transpose(0, 2, 1)
    y_embed = _conv(x, w_assembly, b_assembly).reshape(N, Cr, L).transpose(0, 2, 1)

    hash_buckets = min(L // CHUNK + (L // CHUNK) % 2, 128)
    rot = jax.random.normal(jax.random.key(42), (1, C, N_HASHES, hash_buckets // 2),
                            dtype=x_embed.dtype)
    rot = jnp.broadcast_to(rot, (N, C, N_HASHES, hash_buckets // 2))
    rotated = jnp.einsum('btf,bfhi->bhti', x_embed, rot)
    rotated = jnp.concatenate([rotated, -rotated], axis=-1)
    hash_codes = jnp.argmax(rotated, axis=-1)
    offsets = (jnp.arange(N_HASHES) * hash_buckets).reshape(1, -1, 1)
    hash_codes = (hash_codes + offsets).reshape(N, -1)

    indices = jnp.argsort(hash_codes, axis=-1)
    undo_sort = jnp.argsort(indices, axis=-1)
    mod_indices = indices % L

    x_sorted = jnp.take_along_axis(x_embed, mod_indices[:, :, None], axis=1)
    y_sorted = jnp.take_along_axis(y_embed, mod_indices[:, :, None], axis=1)

    padding = CHUNK - L % CHUNK if L % CHUNK != 0 else 0
    xs = x_sorted.reshape(N * N_HASHES, L, C)
    ys = y_sorted.reshape(N * N_HASHES, L, Cr)
    if padding:
        xs = jnp.concatenate([xs, xs[:, -padding:, :]], axis=1)
        ys = jnp.concatenate([ys, ys[:, -padding:, :]], axis=1)

    ret, bucket_score = _banded_attention(xs, ys)

    if padding:
        ret = ret[:, :-padding, :]
        bucket_score = bucket_score[:, :-padding]
    ret = ret.reshape(N, N_HASHES * L, Cr)
    bucket_score = bucket_score.reshape(N, N_HASHES * L)

    ret = jnp.take_along_axis(ret, undo_sort[:, :, None], axis=1)
    bucket_score = jnp.take_along_axis(bucket_score, undo_sort, axis=1)

    ret = ret.reshape(N, N_HASHES, L, Cr)
    bucket_score = bucket_score.reshape(N, N_HASHES, L, 1)
    probs = jax.nn.softmax(bucket_score, axis=1)
    ret = jnp.sum(ret * probs, axis=1)
    ret = ret.transpose(0, 2, 1).reshape(N, Cr, Hh, Ww) * RES_SCALE + x
    return ret


# speedup vs baseline: 1.0510x; 1.0510x over previous
import functools

import jax
import jax.numpy as jnp
from jax.experimental import pallas as pl

N_HASHES = 4
CHUNK = 144
RES_SCALE = 1.0
REDUCTION = 4


def _conv(x, w, b):
    out = jax.lax.conv_general_dilated(x, w, (1, 1), 'SAME',
                                       dimension_numbers=('NCHW', 'OIHW', 'NCHW'))
    return out + b[None, :, None, None]


def _attn_kernel(xys_ref, out_ref, lse_ref, *, nc, C, Cr):
    k = pl.program_id(1)

    prev = jax.lax.rem(k - 1 + nc, nc)
    nxt = jax.lax.rem(k + 1, nc)

    def load(j):
        return xys_ref[0, pl.ds(j * CHUNK, CHUNK), :]

    t_c = load(k)
    t_p = load(prev)
    t_n = load(nxt)
    q = t_c[:, :C]

    def score_block(t):
        kx = t[:, :C]
        nrm = jnp.sqrt(jnp.sum(kx * kx, axis=1, keepdims=True))
        kn = kx / jnp.maximum(nrm, 5e-05)
        return jax.lax.dot_general(q, kn, (((1,), (1,)), ((), ())),
                                   preferred_element_type=jnp.float32)

    s_c = score_block(t_c)
    s_p = score_block(t_p)
    s_n = score_block(t_n)

    m = jnp.maximum(jnp.maximum(jnp.max(s_c, axis=1), jnp.max(s_p, axis=1)),
                    jnp.max(s_n, axis=1))[:, None]
    p_c = jnp.exp(s_c - m)
    p_p = jnp.exp(s_p - m)
    p_n = jnp.exp(s_n - m)
    denom = (jnp.sum(p_c, axis=1) + jnp.sum(p_p, axis=1)
             + jnp.sum(p_n, axis=1))[:, None]

    def pv(p, t):
        return jax.lax.dot_general(p, t[:, C:], (((1,), (0,)), ((), ())),
                                   preferred_element_type=jnp.float32)

    acc = pv(p_c, t_c) + pv(p_p, t_p) + pv(p_n, t_n)
    out_ref[0, :, :] = acc / denom
    lse_ref[0, :, :] = m + jnp.log(denom)


def _banded_attention(xys, C, Cr):
    H, Lp, _ = xys.shape
    nc = Lp // CHUNK
    kern = functools.partial(_attn_kernel, nc=nc, C=C, Cr=Cr)
    out, lse = pl.pallas_call(
        kern,
        grid=(H, nc),
        in_specs=[
            pl.BlockSpec((1, Lp, C + Cr), lambda h, k: (h, 0, 0)),
        ],
        out_specs=[
            pl.BlockSpec((1, CHUNK, Cr), lambda h, k: (h, k, 0)),
            pl.BlockSpec((1, CHUNK, 1), lambda h, k: (h, k, 0)),
        ],
        out_shape=[
            jax.ShapeDtypeStruct((H, Lp, Cr), jnp.float32),
            jax.ShapeDtypeStruct((H, Lp, 1), jnp.float32),
        ],
    )(xys)
    return out, lse[..., 0]


def kernel(input, w_match, b_match, w_assembly, b_assembly):
    x = input
    N, _, Hh, Ww = x.shape
    L = Hh * Ww
    C = w_match.shape[0]
    Cr = w_assembly.shape[0]

    x_embed = _conv(x, w_match, b_match).reshape(N, C, L).transpose(0, 2, 1)
    y_embed = _conv(x, w_assembly, b_assembly).reshape(N, Cr, L).transpose(0, 2, 1)

    hash_buckets = min(L // CHUNK + (L // CHUNK) % 2, 128)
    rot = jax.random.normal(jax.random.key(42), (1, C, N_HASHES, hash_buckets // 2),
                            dtype=x_embed.dtype)
    rot = jnp.broadcast_to(rot, (N, C, N_HASHES, hash_buckets // 2))
    rotated = jnp.einsum('btf,bfhi->bhti', x_embed, rot)
    rotated = jnp.concatenate([rotated, -rotated], axis=-1)
    hash_codes = jnp.argmax(rotated, axis=-1)
    offsets = (jnp.arange(N_HASHES) * hash_buckets).reshape(1, -1, 1)
    hash_codes = (hash_codes + offsets).reshape(N, -1)

    indices = jnp.argsort(hash_codes, axis=-1)
    undo_sort = jnp.argsort(indices, axis=-1)
    mod_indices = indices % L

    xy_embed = jnp.concatenate([x_embed, y_embed], axis=-1)
    xy_sorted = jnp.take_along_axis(xy_embed, mod_indices[:, :, None], axis=1)

    padding = CHUNK - L % CHUNK if L % CHUNK != 0 else 0
    xys = xy_sorted.reshape(N * N_HASHES, L, C + Cr)
    if padding:
        xys = jnp.concatenate([xys, xys[:, -padding:, :]], axis=1)

    ret, bucket_score = _banded_attention(xys, C, Cr)

    if padding:
        ret = ret[:, :-padding, :]
        bucket_score = bucket_score[:, :-padding]
    ret = ret.reshape(N, N_HASHES * L, Cr)
    bucket_score = bucket_score.reshape(N, N_HASHES * L)

    ret = jnp.take_along_axis(ret, undo_sort[:, :, None], axis=1)
    bucket_score = jnp.take_along_axis(bucket_score, undo_sort, axis=1)

    ret = ret.reshape(N, N_HASHES, L, Cr)
    bucket_score = bucket_score.reshape(N, N_HASHES, L, 1)
    probs = jax.nn.softmax(bucket_score, axis=1)
    ret = jnp.sum(ret * probs, axis=1)
    ret = ret.transpose(0, 2, 1).reshape(N, Cr, Hh, Ww) * RES_SCALE + x
    return ret


# speedup vs baseline: 1.3367x; 1.2718x over previous
import functools

import jax
import jax.numpy as jnp
from jax.experimental import pallas as pl
from jax.experimental.pallas import tpu as pltpu

N_HASHES = 4
CHUNK = 144
RES_SCALE = 1.0
REDUCTION = 4
RANK_TILE = 512


def _conv(x, w, b):
    out = jax.lax.conv_general_dilated(x, w, (1, 1), 'SAME',
                                       dimension_numbers=('NCHW', 'OIHW', 'NCHW'))
    return out + b[None, :, None, None]


def _rank_kernel(codes_ref, tri_ref, r1_ref, hist_ref, carry_ref, *, nb):
    t = pl.program_id(1)

    @pl.when(t == 0)
    def _():
        carry_ref[...] = jnp.zeros_like(carry_ref)

    c = codes_ref[0, :, :]
    lanes = jax.lax.broadcasted_iota(jnp.int32, (1, nb), 1)
    onehot = (c == lanes).astype(jnp.float32)

    cum = jax.lax.dot_general(tri_ref[...], onehot.astype(jnp.bfloat16),
                              (((1,), (0,)), ((), ())),
                              preferred_element_type=jnp.float32)
    carry = carry_ref[0:1, :]
    r1 = jnp.sum(onehot * (cum + carry), axis=1, keepdims=True)
    r1_ref[0, :, :] = r1.astype(jnp.int32)

    new_carry = carry + jnp.sum(onehot, axis=0, keepdims=True)
    carry_ref[0:1, :] = new_carry
    hist_ref[0, :, :] = new_carry


def _pos_kernel(codes_ref, r1_ref, base_ref, pos_ref, *, nb, L):
    h = pl.program_id(0)
    c = codes_ref[0, :, :]
    lanes = jax.lax.broadcasted_iota(jnp.int32, (1, nb), 1)
    onehot = (c == lanes).astype(jnp.float32)
    base = base_ref[0, :, :].astype(jnp.float32)
    b = jnp.sum(onehot * base, axis=1, keepdims=True)
    pos_ref[0, :, :] = (b.astype(jnp.int32) + r1_ref[0, :, :] + h * L)


def _counting_rank(codes, L):
    H, Lpad = codes.shape
    T = RANK_TILE
    nt = Lpad // T
    nb = 128
    codes3 = codes.reshape(H * nt, T, 1)
    ii = jax.lax.broadcasted_iota(jnp.int32, (T, T), 0)
    jj = jax.lax.broadcasted_iota(jnp.int32, (T, T), 1)
    tri = (jj < ii).astype(jnp.bfloat16)

    r1, hist = pl.pallas_call(
        functools.partial(_rank_kernel, nb=nb),
        grid=(H, nt),
        in_specs=[
            pl.BlockSpec((1, T, 1), lambda h, t: (h * nt + t, 0, 0)),
            pl.BlockSpec((T, T), lambda h, t: (0, 0)),
        ],
        out_specs=[
            pl.BlockSpec((1, T, 1), lambda h, t: (h * nt + t, 0, 0)),
            pl.BlockSpec((1, 1, nb), lambda h, t: (h * nt + t, 0, 0)),
        ],
        out_shape=[
            jax.ShapeDtypeStruct((H * nt, T, 1), jnp.int32),
            jax.ShapeDtypeStruct((H * nt, 1, nb), jnp.float32),
        ],
        scratch_shapes=[pltpu.VMEM((8, nb), jnp.float32)],
    )(codes3, tri)

    totals = hist.reshape(H, nt, nb)[:, -1, :]
    bucket_base = (jnp.cumsum(totals, axis=1) - totals).astype(jnp.int32)
    base3 = bucket_base.reshape(H, 1, nb)

    pos = pl.pallas_call(
        functools.partial(_pos_kernel, nb=nb, L=L),
        grid=(H, nt),
        in_specs=[
            pl.BlockSpec((1, T, 1), lambda h, t: (h * nt + t, 0, 0)),
            pl.BlockSpec((1, T, 1), lambda h, t: (h * nt + t, 0, 0)),
            pl.BlockSpec((1, 1, nb), lambda h, t: (h, 0, 0)),
        ],
        out_specs=pl.BlockSpec((1, T, 1), lambda h, t: (h * nt + t, 0, 0)),
        out_shape=jax.ShapeDtypeStruct((H * nt, T, 1), jnp.int32),
    )(codes3, r1, base3)
    return pos.reshape(H, Lpad)


def _attn_kernel(xys_ref, out_ref, lse_ref, *, nc, C, Cr):
    k = pl.program_id(1)

    prev = jax.lax.rem(k - 1 + nc, nc)
    nxt = jax.lax.rem(k + 1, nc)

    def load(j):
        return xys_ref[0, pl.ds(j * CHUNK, CHUNK), :]

    t_c = load(k)
    t_p = load(prev)
    t_n = load(nxt)
    q = t_c[:, :C]

    def score_block(t):
        kx = t[:, :C]
        nrm = jnp.sqrt(jnp.sum(kx * kx, axis=1, keepdims=True))
        kn = kx / jnp.maximum(nrm, 5e-05)
        return jax.lax.dot_general(q, kn, (((1,), (1,)), ((), ())),
                                   preferred_element_type=jnp.float32)

    s_c = score_block(t_c)
    s_p = score_block(t_p)
    s_n = score_block(t_n)

    m = jnp.maximum(jnp.maximum(jnp.max(s_c, axis=1), jnp.max(s_p, axis=1)),
                    jnp.max(s_n, axis=1))[:, None]
    p_c = jnp.exp(s_c - m)
    p_p = jnp.exp(s_p - m)
    p_n = jnp.exp(s_n - m)
    denom = (jnp.sum(p_c, axis=1) + jnp.sum(p_p, axis=1)
             + jnp.sum(p_n, axis=1))[:, None]

    def pv(p, t):
        return jax.lax.dot_general(p, t[:, C:], (((1,), (0,)), ((), ())),
                                   preferred_element_type=jnp.float32)

    acc = pv(p_c, t_c) + pv(p_p, t_p) + pv(p_n, t_n)
    out_ref[0, :, :] = acc / denom
    lse_ref[0, :, :] = m + jnp.log(denom)


def _banded_attention(xys, C, Cr):
    H, Lp, _ = xys.shape
    nc = Lp // CHUNK
    kern = functools.partial(_attn_kernel, nc=nc, C=C, Cr=Cr)
    out, lse = pl.pallas_call(
        kern,
        grid=(H, nc),
        in_specs=[
            pl.BlockSpec((1, Lp, C + Cr), lambda h, k: (h, 0, 0)),
        ],
        out_specs=[
            pl.BlockSpec((1, CHUNK, Cr), lambda h, k: (h, k, 0)),
            pl.BlockSpec((1, CHUNK, 1), lambda h, k: (h, k, 0)),
        ],
        out_shape=[
            jax.ShapeDtypeStruct((H, Lp, Cr), jnp.float32),
            jax.ShapeDtypeStruct((H, Lp, 1), jnp.float32),
        ],
    )(xys)
    return out, lse[..., 0]


def kernel(input, w_match, b_match, w_assembly, b_assembly):
    x = input
    N, _, Hh, Ww = x.shape
    L = Hh * Ww
    C = w_match.shape[0]
    Cr = w_assembly.shape[0]
    H = N_HASHES

    x_embed = _conv(x, w_match, b_match).reshape(N, C, L).transpose(0, 2, 1)
    y_embed = _conv(x, w_assembly, b_assembly).reshape(N, Cr, L).transpose(0, 2, 1)

    hash_buckets = min(L // CHUNK + (L // CHUNK) % 2, 128)
    rot = jax.random.normal(jax.random.key(42), (1, C, H, hash_buckets // 2),
                            dtype=x_embed.dtype)
    rot = jnp.broadcast_to(rot, (N, C, H, hash_buckets // 2))
    rotated = jnp.einsum('btf,bfhi->bhti', x_embed, rot)
    rotated = jnp.concatenate([rotated, -rotated], axis=-1)
    codes = jnp.argmax(rotated, axis=-1).astype(jnp.int32)

    codes2 = codes.reshape(H, L)
    rank_pad = (-L) % RANK_TILE
    if rank_pad:
        codes2 = jnp.concatenate(
            [codes2, jnp.full((H, rank_pad), 128, jnp.int32)], axis=1)
    pos = _counting_rank(codes2, L)[:, :L]
    undo_sort = pos.reshape(N, H * L)

    flat_ids = jnp.broadcast_to(
        jnp.arange(L, dtype=jnp.int32)[None, :], (H, L)).reshape(-1)
    mod_indices = jnp.zeros((H * L,), jnp.int32).at[pos.reshape(-1)].set(
        flat_ids, unique_indices=True).reshape(N, H * L)

    xy_embed = jnp.concatenate([x_embed, y_embed], axis=-1)
    xy_sorted = jnp.take_along_axis(xy_embed, mod_indices[:, :, None], axis=1)

    padding = CHUNK - L % CHUNK if L % CHUNK != 0 else 0
    xys = xy_sorted.reshape(N * H, L, C + Cr)
    if padding:
        xys = jnp.concatenate([xys, xys[:, -padding:, :]], axis=1)

    ret, bucket_score = _banded_attention(xys, C, Cr)

    if padding:
        ret = ret[:, :-padding, :]
        bucket_score = bucket_score[:, :-padding]
    ret = ret.reshape(N, H * L, Cr)
    bucket_score = bucket_score.reshape(N, H * L)

    ret = jnp.take_along_axis(ret, undo_sort[:, :, None], axis=1)
    bucket_score = jnp.take_along_axis(bucket_score, undo_sort, axis=1)

    ret = ret.reshape(N, H, L, Cr)
    bucket_score = bucket_score.reshape(N, H, L, 1)
    probs = jax.nn.softmax(bucket_score, axis=1)
    ret = jnp.sum(ret * probs, axis=1)
    ret = ret.transpose(0, 2, 1).reshape(N, Cr, Hh, Ww) * RES_SCALE + x
    return ret
